# Initial kernel scaffold; baseline (speedup 1.0000x reference)
import functools

import jax
import jax.numpy as jnp
from jax import lax
from jax.experimental import pallas as pl
from jax.experimental.pallas import tpu as pltpu
from jax.experimental.pallas import tpu_sc as plsc

N_NODES = 100000
N_EDGES = 3200000
DIM = 4

NC = 2
NS = 16
NW = NC * NS
E_PER_TILE = N_EDGES // NW
CHUNK = 2000
N_CHUNKS = E_PER_TILE // CHUNK
GROUPS = CHUNK // 16
ROWS_PER_TILE = N_NODES // NS


def _make_pass(transpose: bool):
  mesh = plsc.VectorSubcoreMesh(core_axis_name="c", subcore_axis_name="s")

  @functools.partial(
      pl.kernel,
      mesh=mesh,
      out_type=jax.ShapeDtypeStruct((NC * N_NODES, DIM), jnp.float32),
      scratch_types=[
          pltpu.VMEM_SHARED((N_NODES, DIM), jnp.float32),
          pltpu.VMEM((CHUNK, DIM * DIM), jnp.float32),
          pltpu.VMEM((CHUNK,), jnp.int32),
          pltpu.VMEM((CHUNK,), jnp.int32),
          pltpu.VMEM((CHUNK, DIM), jnp.float32),
          pltpu.VMEM((CHUNK, DIM), jnp.float32),
          pltpu.SemaphoreType.DMA,
      ],
  )
  def pass_kernel(table_hbm, gidx_hbm, sidx_hbm, boo_hbm, zeros_hbm, out_hbm,
                  acc_s, boo_v, gidx_v, sidx_v, rows_v, msg_v, sem):
    c = lax.axis_index("c")
    s = lax.axis_index("s")
    wid = c * NS + s

    r0 = s * ROWS_PER_TILE
    pltpu.sync_copy(zeros_hbm.at[pl.ds(r0, ROWS_PER_TILE)],
                    acc_s.at[pl.ds(r0, ROWS_PER_TILE)])
    plsc.subcore_barrier()

    e0 = wid * E_PER_TILE
    lane = lax.iota(jnp.int32, 16)

    def chunk_body(i, carry):
      base = e0 + i * CHUNK
      pltpu.sync_copy(boo_hbm.at[pl.ds(base, CHUNK)], boo_v)
      pltpu.sync_copy(gidx_hbm.at[pl.ds(base, CHUNK)], gidx_v)
      pltpu.sync_copy(sidx_hbm.at[pl.ds(base, CHUNK)], sidx_v)
      pltpu.async_copy(table_hbm.at[gidx_v], rows_v, sem).wait()

      def group_body(g, carry2):
        rows = g * 16 + lane
        xd = [
            plsc.load_gather(rows_v, [rows, jnp.full((16,), d, jnp.int32)])
            for d in range(DIM)
        ]
        for k in range(DIM):
          acc = None
          for d in range(DIM):
            off = d * DIM + k if transpose else k * DIM + d
            b = plsc.load_gather(boo_v, [rows, jnp.full((16,), off, jnp.int32)])
            acc = b * xd[d] if acc is None else acc + b * xd[d]
        plsc.store_scatter(msg_v, [rows, jnp.full((16,), k, jnp.int32)], acc)
        return carry2

      lax.fori_loop(0, GROUPS, group_body, 0)
      pltpu.sync_copy(msg_v, acc_s.at[sidx_v], add=True)
      return carry

    lax.fori_loop(0, N_CHUNKS, chunk_body, 0)
    plsc.subcore_barrier()
    pltpu.sync_copy(acc_s.at[pl.ds(r0, ROWS_PER_TILE)],
                    out_hbm.at[pl.ds(c * N_NODES + r0, ROWS_PER_TILE)])

  return pass_kernel


_pass_t = _make_pass(transpose=True)
_pass_n = _make_pass(transpose=False)

_FLAT = N_NODES * DIM
_CB = 16000


def _combine_body(p_ref, o_ref):
  o_ref[...] = p_ref[0] + p_ref[1]


def _combine(partials):
  flat = partials.reshape(NC, _FLAT)
  out = pl.pallas_call(
      _combine_body,
      grid=(_FLAT // _CB,),
      in_specs=[pl.BlockSpec((NC, _CB), lambda i: (0, i))],
      out_specs=pl.BlockSpec((_CB,), lambda i: (i,)),
      out_shape=jax.ShapeDtypeStruct((_FLAT,), jnp.float32),
  )(flat)
  return out.reshape(N_NODES, DIM)


def kernel(x, edge_index, boo_values):
  src = edge_index[0].astype(jnp.int32)
  dst = edge_index[1].astype(jnp.int32)
  boo = boo_values.reshape(N_EDGES, DIM * DIM)
  zeros = jnp.zeros((N_NODES, DIM), jnp.float32)

  p1 = _pass_t(x, src, dst, boo, zeros)
  lt_x = _combine(p1)
  p2 = _pass_n(lt_x, dst, src, boo, zeros)
  return _combine(p2)


# speedup vs baseline: 16.8203x; 16.8203x over previous
import functools

import jax
import jax.numpy as jnp
from jax import lax
from jax.experimental import pallas as pl
from jax.experimental.pallas import tpu as pltpu
from jax.experimental.pallas import tpu_sc as plsc

N_NODES = 100000
N_EDGES = 3200000
DIM = 4
ROW_W = 8

NC = 2
NS = 16
NW = NC * NS
E_PER_TILE = N_EDGES // NW
CHUNK = 2000
N_CHUNKS = E_PER_TILE // CHUNK
GROUPS = CHUNK // 16
N_PAD = 100096
ROWS_PER_TILE = N_PAD // NS


def _make_pass(transpose: bool):
  mesh = plsc.VectorSubcoreMesh(core_axis_name="c", subcore_axis_name="s")

  @functools.partial(
      pl.kernel,
      mesh=mesh,
      compiler_params=pltpu.CompilerParams(
          needs_layout_passes=False, use_tc_tiling_on_sc=False),
      out_type=jax.ShapeDtypeStruct((NC * N_PAD, ROW_W), jnp.float32),
      scratch_types=[
          pltpu.VMEM_SHARED((N_PAD, ROW_W), jnp.float32),
          pltpu.VMEM((CHUNK, DIM * DIM), jnp.float32),
          pltpu.VMEM((CHUNK,), jnp.int32),
          pltpu.VMEM((CHUNK,), jnp.int32),
          pltpu.VMEM((CHUNK, ROW_W), jnp.float32),
          pltpu.VMEM((CHUNK, ROW_W), jnp.float32),
          pltpu.SemaphoreType.DMA,
      ],
  )
  def pass_kernel(table_hbm, gidx_hbm, sidx_hbm, boo_hbm, zeros_hbm, out_hbm,
                  acc_s, boo_v, gidx_v, sidx_v, rows_v, msg_v, sem):
    c = lax.axis_index("c")
    s = lax.axis_index("s")
    wid = c * NS + s

    r0 = s * ROWS_PER_TILE
    pltpu.sync_copy(zeros_hbm.at[pl.ds(r0, ROWS_PER_TILE)],
                    acc_s.at[pl.ds(r0, ROWS_PER_TILE)])
    pltpu.sync_copy(zeros_hbm.at[pl.ds(0, CHUNK)], msg_v)
    plsc.subcore_barrier()

    e0 = wid * E_PER_TILE
    lane = lax.iota(jnp.int32, 16)

    def chunk_body(i, carry):
      base = e0 + i * CHUNK
      pltpu.sync_copy(boo_hbm.at[pl.ds(base, CHUNK)], boo_v)
      pltpu.sync_copy(gidx_hbm.at[pl.ds(base, CHUNK)], gidx_v)
      pltpu.sync_copy(sidx_hbm.at[pl.ds(base, CHUNK)], sidx_v)
      pltpu.async_copy(table_hbm.at[gidx_v], rows_v, sem).wait()

      def group_body(g, carry2):
        rows = g * 16 + lane
        xd = [
            plsc.load_gather(rows_v, [rows, jnp.full((16,), d, jnp.int32)])
            for d in range(DIM)
        ]
        for k in range(DIM):
          acc = None
          for d in range(DIM):
            off = d * DIM + k if transpose else k * DIM + d
            b = plsc.load_gather(boo_v, [rows, jnp.full((16,), off, jnp.int32)])
            acc = b * xd[d] if acc is None else acc + b * xd[d]
          plsc.store_scatter(msg_v, [rows, jnp.full((16,), k, jnp.int32)], acc)
        return carry2

      lax.fori_loop(0, GROUPS, group_body, 0)
      pltpu.sync_copy(msg_v, acc_s.at[sidx_v], add=True)
      return carry

    lax.fori_loop(0, N_CHUNKS, chunk_body, 0)
    plsc.subcore_barrier()
    pltpu.sync_copy(acc_s.at[pl.ds(r0, ROWS_PER_TILE)],
                    out_hbm.at[pl.ds(c * N_PAD + r0, ROWS_PER_TILE)])

  return pass_kernel


_pass_t = _make_pass(transpose=True)
_pass_n = _make_pass(transpose=False)

_CROWS = N_PAD * ROW_W // 128
_CB = 272


def _combine_body(p_ref, o_ref):
  o_ref[...] = p_ref[0] + p_ref[1]


def _combine(partials):
  flat = partials.reshape(NC, _CROWS, 128)
  out = pl.pallas_call(
      _combine_body,
      grid=(_CROWS // _CB,),
      in_specs=[pl.BlockSpec((NC, _CB, 128), lambda i: (0, i, 0))],
      out_specs=pl.BlockSpec((_CB, 128), lambda i: (i, 0)),
      out_shape=jax.ShapeDtypeStruct((_CROWS, 128), jnp.float32),
  )(flat)
  return out.reshape(N_PAD, ROW_W)


def kernel(x, edge_index, boo_values):
  src = edge_index[0].astype(jnp.int32)
  dst = edge_index[1].astype(jnp.int32)
  boo = boo_values.reshape(N_EDGES, DIM * DIM)
  zeros = jnp.zeros((N_PAD, ROW_W), jnp.float32)
  x_pad = zeros.at[:N_NODES, :DIM].set(x)

  p1 = _pass_t(x_pad, src, dst, boo, zeros)
  lt_x = _combine(p1)
  p2 = _pass_n(lt_x, dst, src, boo, zeros)
  return _combine(p2)[:N_NODES, :DIM]


# speedup vs baseline: 17.8894x; 1.0636x over previous
import functools

import jax
import jax.numpy as jnp
from jax import lax
from jax.experimental import pallas as pl
from jax.experimental.pallas import tpu as pltpu
from jax.experimental.pallas import tpu_sc as plsc

N_NODES = 100000
N_EDGES = 3200000
DIM = 4
ROW_W = 8

NC = 2
NS = 16
NW = NC * NS
E_PER_TILE = N_EDGES // NW
CHUNK = 800
N_CHUNKS = E_PER_TILE // CHUNK
GROUPS = CHUNK // 16
N_PAD = 100096
ROWS_PER_TILE = N_PAD // NS


def _make_pass(transpose: bool):
  mesh = plsc.VectorSubcoreMesh(core_axis_name="c", subcore_axis_name="s")

  @functools.partial(
      pl.kernel,
      mesh=mesh,
      compiler_params=pltpu.CompilerParams(
          needs_layout_passes=False, use_tc_tiling_on_sc=False),
      out_type=jax.ShapeDtypeStruct((NC * N_PAD, ROW_W), jnp.float32),
      scratch_types=[
          pltpu.VMEM_SHARED((N_PAD, ROW_W), jnp.float32),
          pltpu.VMEM((CHUNK, DIM * DIM), jnp.float32),
          pltpu.VMEM((CHUNK, DIM * DIM), jnp.float32),
          pltpu.VMEM((CHUNK,), jnp.int32),
          pltpu.VMEM((CHUNK,), jnp.int32),
          pltpu.VMEM((CHUNK,), jnp.int32),
          pltpu.VMEM((CHUNK,), jnp.int32),
          pltpu.VMEM((CHUNK, ROW_W), jnp.float32),
          pltpu.VMEM((CHUNK, ROW_W), jnp.float32),
          pltpu.VMEM((CHUNK, ROW_W), jnp.float32),
          pltpu.SemaphoreType.DMA,
          pltpu.SemaphoreType.DMA,
          pltpu.SemaphoreType.DMA,
      ],
  )
  def pass_kernel(table_hbm, gidx_hbm, sidx_hbm, boo_hbm, zeros_hbm, out_hbm,
                  acc_s, boo0, boo1, gidx0, gidx1, sidx0, sidx1,
                  rows0, rows1, msg_v, sem0, sem1, semg):
    c = lax.axis_index("c")
    s = lax.axis_index("s")
    wid = c * NS + s
    sets = ((boo0, gidx0, sidx0, rows0, sem0),
            (boo1, gidx1, sidx1, rows1, sem1))

    r0 = s * ROWS_PER_TILE
    pltpu.sync_copy(zeros_hbm.at[pl.ds(r0, ROWS_PER_TILE)],
                    acc_s.at[pl.ds(r0, ROWS_PER_TILE)])
    pltpu.sync_copy(zeros_hbm.at[pl.ds(0, CHUNK)], msg_v)
    plsc.subcore_barrier()

    e0 = wid * E_PER_TILE
    lane = lax.iota(jnp.int32, 16)

    def start_dmas(p, base):
      boo_v, gidx_v, sidx_v, _, sem = sets[p]
      pltpu.async_copy(boo_hbm.at[pl.ds(base, CHUNK)], boo_v, sem)
      pltpu.async_copy(gidx_hbm.at[pl.ds(base, CHUNK)], gidx_v, sem)
      pltpu.async_copy(sidx_hbm.at[pl.ds(base, CHUNK)], sidx_v, sem)

    def wait_dmas(p, base):
      boo_v, gidx_v, sidx_v, _, sem = sets[p]
      pltpu.make_async_copy(boo_hbm.at[pl.ds(base, CHUNK)], boo_v, sem).wait()
      pltpu.make_async_copy(gidx_hbm.at[pl.ds(base, CHUNK)], gidx_v, sem).wait()
      pltpu.make_async_copy(sidx_hbm.at[pl.ds(base, CHUNK)], sidx_v, sem).wait()

    def process(p, base):
      boo_v, gidx_v, sidx_v, rows_v, _ = sets[p]
      wait_dmas(p, base)
      pltpu.async_copy(table_hbm.at[gidx_v], rows_v, semg).wait()

      def group_body(g, carry2):
        rows = g * 16 + lane
        xd = [
            plsc.load_gather(rows_v, [rows, jnp.full((16,), d, jnp.int32)])
            for d in range(DIM)
        ]
        for k in range(DIM):
          acc = None
          for d in range(DIM):
            off = d * DIM + k if transpose else k * DIM + d
            b = plsc.load_gather(boo_v, [rows, jnp.full((16,), off, jnp.int32)])
            acc = b * xd[d] if acc is None else acc + b * xd[d]
          plsc.store_scatter(msg_v, [rows, jnp.full((16,), k, jnp.int32)], acc)
        return carry2

      lax.fori_loop(0, GROUPS, group_body, 0)
      pltpu.sync_copy(msg_v, acc_s.at[sidx_v], add=True)

    start_dmas(0, e0)
    start_dmas(1, e0 + CHUNK)

    def pair_body(i2, carry):
      base = e0 + 2 * i2 * CHUNK
      for p in (0, 1):
        process(p, base + p * CHUNK)
        nxt = base + (p + 2) * CHUNK

        @pl.when(2 * i2 + p + 2 < N_CHUNKS)
        def _():
          start_dmas(p, nxt)
      return carry

    lax.fori_loop(0, N_CHUNKS // 2, pair_body, 0)
    if N_CHUNKS % 2:
      process(0, e0 + (N_CHUNKS - 1) * CHUNK)
    plsc.subcore_barrier()
    pltpu.sync_copy(acc_s.at[pl.ds(r0, ROWS_PER_TILE)],
                    out_hbm.at[pl.ds(c * N_PAD + r0, ROWS_PER_TILE)])

  return pass_kernel


_pass_t = _make_pass(transpose=True)
_pass_n = _make_pass(transpose=False)

_CROWS = N_PAD * ROW_W // 128
_CB = 272


def _combine_body(p_ref, o_ref):
  o_ref[...] = p_ref[0] + p_ref[1]


def _combine(partials):
  flat = partials.reshape(NC, _CROWS, 128)
  out = pl.pallas_call(
      _combine_body,
      grid=(_CROWS // _CB,),
      in_specs=[pl.BlockSpec((NC, _CB, 128), lambda i: (0, i, 0))],
      out_specs=pl.BlockSpec((_CB, 128), lambda i: (i, 0)),
      out_shape=jax.ShapeDtypeStruct((_CROWS, 128), jnp.float32),
  )(flat)
  return out.reshape(N_PAD, ROW_W)


def kernel(x, edge_index, boo_values):
  src = edge_index[0].astype(jnp.int32)
  dst = edge_index[1].astype(jnp.int32)
  boo = boo_values.reshape(N_EDGES, DIM * DIM)
  zeros = jnp.zeros((N_PAD, ROW_W), jnp.float32)
  x_pad = zeros.at[:N_NODES, :DIM].set(x)

  p1 = _pass_t(x_pad, src, dst, boo, zeros)
  lt_x = _combine(p1)
  p2 = _pass_n(lt_x, dst, src, boo, zeros)
  return _combine(p2)[:N_NODES, :DIM]
